# baseline (speedup 1.0000x reference)
import functools

import jax
import jax.numpy as jnp
from jax import lax
from jax.experimental import pallas as pl
from jax.experimental.pallas import tpu as pltpu
from jax.experimental.pallas import tpu_sc as plsc

N_NODES = 10000
NPAD = 10240
NT = 16
NSC = 2
CK = 128
FSL = 128
ACC_ROWS = 10016
RPT = 632
LAST = ACC_ROWS - (NT - 1) * RPT
EPS = 1e-5


def _zero_stripe(zsrc, acc, base, nrows):
    off = 0
    while off < nrows:
        step = min(CK, nrows - off)
        src = zsrc if step == CK else zsrc.at[pl.ds(0, step)]
        pltpu.sync_copy(src, acc.at[pl.ds(base + off, step)])
        off += step


def _per_stripe(tid, fn):
    @pl.when(tid < NT - 1)
    def _():
        fn(tid * RPT, RPT)

    @pl.when(tid == NT - 1)
    def _():
        fn((NT - 1) * RPT, LAST)


def _fill(ref, nrows, ncols, value):
    vals = jnp.full((16,), value, jnp.float32)

    def row(i, _):
        def col(k, _):
            ref[i, pl.ds(k * 16, 16)] = vals
            return 0
        return lax.fori_loop(0, ncols // 16, col, 0)

    lax.fori_loop(0, nrows, row, 0)


def _deg_body(nch, dst3, degp, didx, ones_v, acc):
    c = lax.axis_index("c")
    tid = lax.axis_index("s")

    _fill(ones_v, CK, FSL, 0.0)
    _per_stripe(tid, lambda base, nr: _zero_stripe(ones_v, acc, base, nr))
    _fill(ones_v, CK, FSL, 1.0)
    pltpu.sync_copy(dst3.at[tid], didx)
    plsc.subcore_barrier()

    half = nch // 2

    def body(j, _):
        pltpu.sync_copy(ones_v, acc.at[didx.at[j]], add=True)
        return 0

    lax.fori_loop(c * half, (c + 1) * half, body, 0)
    plsc.subcore_barrier()

    def wb(base, nr):
        pltpu.sync_copy(acc.at[pl.ds(base, nr)],
                        degp.at[c, pl.ds(base, nr)])

    _per_stripe(tid, wb)


def _make_deg(nch):
    mesh = plsc.VectorSubcoreMesh(core_axis_name="c", subcore_axis_name="s",
                                  num_cores=NSC, num_subcores=NT)
    return pl.kernel(
        functools.partial(_deg_body, nch),
        out_type=jax.ShapeDtypeStruct((NSC, ACC_ROWS, FSL), jnp.float32),
        mesh=mesh,
        scratch_types=[
            pltpu.VMEM((nch, CK), jnp.int32),
            pltpu.VMEM((CK, FSL), jnp.float32),
            pltpu.VMEM_SHARED((ACC_ROWS, FSL), jnp.float32),
        ],
    )


BLK = 4


def _agg_body(nblk, hs4, idx6, agg4, idxb, rows0, rows1, rows2,
              acc, gsem0, gsem1, gsem2):
    c = lax.axis_index("c")
    tid = lax.axis_index("s")
    rows = (rows0, rows1, rows2)
    gsems = (gsem0, gsem1, gsem2)

    for p in range(2):
        f = c * 2 + p
        _fill(rows0, CK, FSL, 0.0)
        _per_stripe(tid, lambda base, nr: _zero_stripe(rows0, acc, base, nr))
        plsc.subcore_barrier()

        tbl = hs4.at[f]

        def block(blk, _):
            pltpu.sync_copy(idx6.at[tid, blk], idxb)
            pltpu.async_copy(tbl.at[idxb.at[0]], rows[0], gsems[0])
            pltpu.async_copy(tbl.at[idxb.at[1]], rows[1], gsems[1])
            for j in range(BLK):
                b = j % 3
                if j + 2 < BLK:
                    nb = (j + 2) % 3
                    pltpu.async_copy(tbl.at[idxb.at[j + 2]], rows[nb],
                                     gsems[nb])
                pltpu.make_async_copy(tbl.at[idxb.at[j]], rows[b],
                                      gsems[b]).wait()
                pltpu.sync_copy(rows[b], acc.at[idxb.at[BLK + j]], add=True)
            return 0

        lax.fori_loop(0, nblk, block, 0)
        plsc.subcore_barrier()

        def wb(base, nr):
            pltpu.sync_copy(acc.at[pl.ds(base, nr)],
                            agg4.at[f, pl.ds(base, nr)])

        _per_stripe(tid, wb)


def _make_agg(nblk):
    mesh = plsc.VectorSubcoreMesh(core_axis_name="c", subcore_axis_name="s",
                                  num_cores=NSC, num_subcores=NT)
    return pl.kernel(
        functools.partial(_agg_body, nblk),
        out_type=jax.ShapeDtypeStruct((4, NPAD, FSL), jnp.float32),
        mesh=mesh,
        scratch_types=[
            pltpu.VMEM((2 * BLK, CK), jnp.int32),
            pltpu.VMEM((CK, FSL), jnp.float32),
            pltpu.VMEM((CK, FSL), jnp.float32),
            pltpu.VMEM((CK, FSL), jnp.float32),
            pltpu.VMEM_SHARED((ACC_ROWS, FSL), jnp.float32),
            pltpu.SemaphoreType.DMA,
            pltpu.SemaphoreType.DMA,
            pltpu.SemaphoreType.DMA,
        ],
    )


def _dinv_of(deg_ref):
    deg = deg_ref[0, 0]
    return jnp.where(deg > 0, lax.rsqrt(deg), 0.0)[:, None]


BNM = 512
BNC = 256


def _mm_kernel(x_ref, w_ref, r_ref, rb_ref, deg_ref, hs_ref, res_ref):
    dinv = _dinv_of(deg_ref)
    xb = x_ref[...].astype(jnp.bfloat16)
    h = jnp.dot(xb, w_ref[...].astype(jnp.bfloat16),
                preferred_element_type=jnp.float32)
    hs = h * dinv
    hs_ref[0] = hs[:, :FSL]
    hs_ref[1] = hs[:, FSL:]
    res = jnp.dot(xb, r_ref[...].astype(jnp.bfloat16),
                  preferred_element_type=jnp.float32)
    res_ref[...] = res + rb_ref[0]


def _mm_call(x, w, r, rb, deg2):
    din = x.shape[1]
    nb = NPAD // BNM
    return pl.pallas_call(
        _mm_kernel,
        grid=(2, nb),
        in_specs=[
            pl.BlockSpec((BNM, din), lambda j, i: (i, 0)),
            pl.BlockSpec((din, 2 * FSL), lambda j, i: (0, j)),
            pl.BlockSpec((din, 2 * FSL), lambda j, i: (0, j)),
            pl.BlockSpec((1, 2 * FSL), lambda j, i: (0, j)),
            pl.BlockSpec((1, 1, BNM), lambda j, i: (i, 0, 0)),
        ],
        out_specs=[
            pl.BlockSpec((2, BNM, FSL), lambda j, i: (j, i, 0)),
            pl.BlockSpec((BNM, 2 * FSL), lambda j, i: (i, j)),
        ],
        out_shape=[
            jax.ShapeDtypeStruct((4, NPAD, FSL), jnp.float32),
            jax.ShapeDtypeStruct((NPAD, 4 * FSL), jnp.float32),
        ],
    )(x, w, r, rb, deg2)


def _combine_kernel(agg_ref, res_ref, deg_ref, b_ref, g_ref, be_ref, out_ref):
    dinv = _dinv_of(deg_ref)
    hs = []
    tot = jnp.zeros((BNC, 1), jnp.float32)
    for f in range(4):
        hf = (agg_ref[f] * dinv + b_ref[0, pl.ds(f * FSL, FSL)]
              + res_ref[:, pl.ds(f * FSL, FSL)])
        hs.append(hf)
        tot = tot + jnp.sum(hf, axis=-1, keepdims=True)
    m = tot * (1.0 / (4 * FSL))
    vtot = jnp.zeros((BNC, 1), jnp.float32)
    for f in range(4):
        vtot = vtot + jnp.sum((hs[f] - m) ** 2, axis=-1, keepdims=True)
    inv = lax.rsqrt(vtot * (1.0 / (4 * FSL)) + EPS)
    for f in range(4):
        nf = (hs[f] - m) * inv * g_ref[0, pl.ds(f * FSL, FSL)] \
            + be_ref[0, pl.ds(f * FSL, FSL)]
        out_ref[:, pl.ds(f * FSL, FSL)] = jnp.maximum(nf, 0.0)


def _combine_call(agg4, res, deg2, b, g, be):
    nb = NPAD // BNC
    return pl.pallas_call(
        _combine_kernel,
        grid=(nb,),
        in_specs=[
            pl.BlockSpec((4, BNC, FSL), lambda i: (0, i, 0)),
            pl.BlockSpec((BNC, 4 * FSL), lambda i: (i, 0)),
            pl.BlockSpec((1, 1, BNC), lambda i: (i, 0, 0)),
            pl.BlockSpec((1, 4 * FSL), lambda i: (0, 0)),
            pl.BlockSpec((1, 4 * FSL), lambda i: (0, 0)),
            pl.BlockSpec((1, 4 * FSL), lambda i: (0, 0)),
        ],
        out_specs=pl.BlockSpec((BNC, 4 * FSL), lambda i: (i, 0)),
        out_shape=jax.ShapeDtypeStruct((NPAD, 4 * FSL), jnp.float32),
    )(agg4, res, deg2, b, g, be)


def _out_kernel(x_ref, w_ref, b_ref, out_ref):
    h = jnp.dot(x_ref[...].astype(jnp.bfloat16),
                w_ref[...].astype(jnp.bfloat16),
                preferred_element_type=jnp.float32)
    out_ref[...] = h + b_ref[0]


def _out_call(x, w, b):
    dout = w.shape[1]
    nb = NPAD // BNC
    return pl.pallas_call(
        _out_kernel,
        grid=(nb,),
        in_specs=[
            pl.BlockSpec((BNC, x.shape[1]), lambda i: (i, 0)),
            pl.BlockSpec((x.shape[1], dout), lambda i: (0, 0)),
            pl.BlockSpec((1, dout), lambda i: (0, 0)),
        ],
        out_specs=pl.BlockSpec((BNC, dout), lambda i: (i, 0)),
        out_shape=jax.ShapeDtypeStruct((NPAD, dout), jnp.float32),
    )(x, w, b)


def kernel(x, edge_index, W1, b1, W2, b2, W3, b3, R1, rb1, R2, rb2, R3, rb3,
           g1, be1, g2, be2, g3, be3, Wout, bout):
    n = x.shape[0]
    e = edge_index.shape[1]
    loops = jnp.arange(n, dtype=edge_index.dtype)
    src = jnp.concatenate([edge_index[0], loops])
    dst = jnp.concatenate([edge_index[1], loops])
    etot = e + n

    nch = -(-etot // (NT * CK))
    nch = -(-nch // (2 * BLK)) * (2 * BLK)
    epad = NT * nch * CK
    src3 = jnp.pad(src, (0, epad - etot),
                   constant_values=n + 1).reshape(NT, nch, CK).astype(jnp.int32)
    dst3 = jnp.pad(dst, (0, epad - etot),
                   constant_values=n + 1).reshape(NT, nch, CK).astype(jnp.int32)
    nblk = nch // BLK
    idx6 = jnp.concatenate([src3.reshape(NT, nblk, BLK, CK),
                            dst3.reshape(NT, nblk, BLK, CK)], axis=2)

    xp = jnp.pad(x, ((0, NPAD - n), (0, 0)))

    degp = _make_deg(nch)(dst3)
    deg = jnp.pad(degp[0, :, 0] + degp[1, :, 0], (0, NPAD - ACC_ROWS))
    deg_mm = deg.reshape(NPAD // BNM, 1, BNM)
    deg_cb = deg.reshape(NPAD // BNC, 1, BNC)

    agg_fn = _make_agg(nblk)

    h = xp
    layers = [(W1, b1, R1, rb1, g1, be1),
              (W2, b2, R2, rb2, g2, be2),
              (W3, b3, R3, rb3, g3, be3)]
    for (W, b, R, rb, g, be) in layers:
        hs4, res = _mm_call(h, W, R, rb[None, :], deg_mm)
        agg4 = agg_fn(hs4, idx6)
        h = _combine_call(agg4, res, deg_cb, b[None, :], g[None, :],
                          be[None, :])

    out = _out_call(h, Wout, bout[None, :])
    return out[:n]


# speedup vs baseline: 2.6403x; 2.6403x over previous
import functools

import jax
import jax.numpy as jnp
from jax import lax
from jax.experimental import pallas as pl
from jax.experimental.pallas import tpu as pltpu
from jax.experimental.pallas import tpu_sc as plsc

N_NODES = 10000
NPAD = 10240
NT = 16
NSC = 2
CK = 128
FSL = 128
RPT = NPAD // NT
EPS = 1e-5


def _zero_stripe(zsrc, acc, base):
    for k in range(RPT // CK):
        pltpu.sync_copy(zsrc, acc.at[pl.ds(base + k * CK, CK)])


def _fill(ref, nrows, ncols, value):
    vals = jnp.full((16,), value, jnp.float32)

    def row(i, _):
        def col(k, _):
            ref[i, pl.ds(k * 16, 16)] = vals
            return 0
        return lax.fori_loop(0, ncols // 16, col, 0)

    lax.fori_loop(0, nrows, row, 0)


def _deg_body(nch, dst3, degp, didx, ones_v, acc):
    c = lax.axis_index("c")
    tid = lax.axis_index("s")
    stripe = tid * RPT

    _fill(ones_v, CK, FSL, 0.0)
    _zero_stripe(ones_v, acc, stripe)
    _fill(ones_v, CK, FSL, 1.0)
    pltpu.sync_copy(dst3.at[tid], didx)
    plsc.subcore_barrier()

    half = nch // 2

    def body(j, _):
        pltpu.sync_copy(ones_v, acc.at[didx.at[j]], add=True)
        return 0

    lax.fori_loop(c * half, (c + 1) * half, body, 0)
    plsc.subcore_barrier()
    pltpu.sync_copy(acc.at[pl.ds(stripe, RPT)],
                    degp.at[c, pl.ds(stripe, RPT)])


def _make_deg(nch):
    mesh = plsc.VectorSubcoreMesh(core_axis_name="c", subcore_axis_name="s",
                                  num_cores=NSC, num_subcores=NT)
    return pl.kernel(
        functools.partial(_deg_body, nch),
        out_type=jax.ShapeDtypeStruct((NSC, NPAD, FSL), jnp.float32),
        mesh=mesh,
        scratch_types=[
            pltpu.VMEM((nch, CK), jnp.int32),
            pltpu.VMEM((CK, FSL), jnp.float32),
            pltpu.VMEM_SHARED((NPAD, FSL), jnp.float32),
        ],
    )


BLK = 12


def _agg_body(nblk, hs4, src4, dst4, agg4, sidxb, didxb, rows0, rows1,
              acc, gsem0, gsem1):
    c = lax.axis_index("c")
    tid = lax.axis_index("s")
    stripe = tid * RPT
    rows = (rows0, rows1)
    gsems = (gsem0, gsem1)

    for p in range(2):
        f = c * 2 + p
        _fill(rows0, CK, FSL, 0.0)
        _zero_stripe(rows0, acc, stripe)
        plsc.subcore_barrier()

        tbl = hs4.at[f]

        def block(blk, _):
            pltpu.sync_copy(src4.at[tid, blk], sidxb)
            pltpu.sync_copy(dst4.at[tid, blk], didxb)
            pltpu.async_copy(tbl.at[sidxb.at[0]], rows[0], gsems[0])
            for j in range(BLK):
                b = j % 2
                ob = 1 - b
                if j + 1 < BLK:
                    pltpu.async_copy(tbl.at[sidxb.at[j + 1]], rows[ob],
                                     gsems[ob])
                pltpu.make_async_copy(tbl.at[sidxb.at[j]], rows[b],
                                      gsems[b]).wait()
                pltpu.sync_copy(rows[b], acc.at[didxb.at[j]], add=True)
            return 0

        lax.fori_loop(0, nblk, block, 0)
        plsc.subcore_barrier()
        pltpu.sync_copy(acc.at[pl.ds(stripe, RPT)],
                        agg4.at[f, pl.ds(stripe, RPT)])


def _make_agg(nblk):
    mesh = plsc.VectorSubcoreMesh(core_axis_name="c", subcore_axis_name="s",
                                  num_cores=NSC, num_subcores=NT)
    return pl.kernel(
        functools.partial(_agg_body, nblk),
        out_type=jax.ShapeDtypeStruct((4, NPAD, FSL), jnp.float32),
        mesh=mesh,
        scratch_types=[
            pltpu.VMEM((BLK, CK), jnp.int32),
            pltpu.VMEM((BLK, CK), jnp.int32),
            pltpu.VMEM((CK, FSL), jnp.float32),
            pltpu.VMEM((CK, FSL), jnp.float32),
            pltpu.VMEM_SHARED((NPAD, FSL), jnp.float32),
            pltpu.SemaphoreType.DMA,
            pltpu.SemaphoreType.DMA,
        ],
    )


def _dinv_of(deg_ref):
    deg = deg_ref[0, 0]
    return jnp.where(deg > 0, lax.rsqrt(deg), 0.0)[:, None]


BNM = 512
BNC = 256


def _mm_kernel(x_ref, w_ref, r_ref, rb_ref, deg_ref, hs_ref, res_ref):
    dinv = _dinv_of(deg_ref)
    xb = x_ref[...].astype(jnp.bfloat16)
    h = jnp.dot(xb, w_ref[...].astype(jnp.bfloat16),
                preferred_element_type=jnp.float32)
    hs = h * dinv
    hs_ref[0] = hs[:, :FSL]
    hs_ref[1] = hs[:, FSL:]
    res = jnp.dot(xb, r_ref[...].astype(jnp.bfloat16),
                  preferred_element_type=jnp.float32)
    res_ref[...] = res + rb_ref[0]


def _mm_call(x, w, r, rb, deg2):
    din = x.shape[1]
    nb = NPAD // BNM
    return pl.pallas_call(
        _mm_kernel,
        grid=(2, nb),
        in_specs=[
            pl.BlockSpec((BNM, din), lambda j, i: (i, 0)),
            pl.BlockSpec((din, 2 * FSL), lambda j, i: (0, j)),
            pl.BlockSpec((din, 2 * FSL), lambda j, i: (0, j)),
            pl.BlockSpec((1, 2 * FSL), lambda j, i: (0, j)),
            pl.BlockSpec((1, 1, BNM), lambda j, i: (i, 0, 0)),
        ],
        out_specs=[
            pl.BlockSpec((2, BNM, FSL), lambda j, i: (j, i, 0)),
            pl.BlockSpec((BNM, 2 * FSL), lambda j, i: (i, j)),
        ],
        out_shape=[
            jax.ShapeDtypeStruct((4, NPAD, FSL), jnp.float32),
            jax.ShapeDtypeStruct((NPAD, 4 * FSL), jnp.float32),
        ],
    )(x, w, r, rb, deg2)


def _combine_kernel(agg_ref, res_ref, deg_ref, b_ref, g_ref, be_ref, out_ref):
    dinv = _dinv_of(deg_ref)
    hs = []
    tot = jnp.zeros((BNC, 1), jnp.float32)
    for f in range(4):
        hf = (agg_ref[f] * dinv + b_ref[0, pl.ds(f * FSL, FSL)]
              + res_ref[:, pl.ds(f * FSL, FSL)])
        hs.append(hf)
        tot = tot + jnp.sum(hf, axis=-1, keepdims=True)
    m = tot * (1.0 / (4 * FSL))
    vtot = jnp.zeros((BNC, 1), jnp.float32)
    for f in range(4):
        vtot = vtot + jnp.sum((hs[f] - m) ** 2, axis=-1, keepdims=True)
    inv = lax.rsqrt(vtot * (1.0 / (4 * FSL)) + EPS)
    for f in range(4):
        nf = (hs[f] - m) * inv * g_ref[0, pl.ds(f * FSL, FSL)] \
            + be_ref[0, pl.ds(f * FSL, FSL)]
        out_ref[:, pl.ds(f * FSL, FSL)] = jnp.maximum(nf, 0.0)


def _combine_call(agg4, res, deg2, b, g, be):
    nb = NPAD // BNC
    return pl.pallas_call(
        _combine_kernel,
        grid=(nb,),
        in_specs=[
            pl.BlockSpec((4, BNC, FSL), lambda i: (0, i, 0)),
            pl.BlockSpec((BNC, 4 * FSL), lambda i: (i, 0)),
            pl.BlockSpec((1, 1, BNC), lambda i: (i, 0, 0)),
            pl.BlockSpec((1, 4 * FSL), lambda i: (0, 0)),
            pl.BlockSpec((1, 4 * FSL), lambda i: (0, 0)),
            pl.BlockSpec((1, 4 * FSL), lambda i: (0, 0)),
        ],
        out_specs=pl.BlockSpec((BNC, 4 * FSL), lambda i: (i, 0)),
        out_shape=jax.ShapeDtypeStruct((NPAD, 4 * FSL), jnp.float32),
    )(agg4, res, deg2, b, g, be)


def _out_kernel(x_ref, w_ref, b_ref, out_ref):
    h = jnp.dot(x_ref[...].astype(jnp.bfloat16),
                w_ref[...].astype(jnp.bfloat16),
                preferred_element_type=jnp.float32)
    out_ref[...] = h + b_ref[0]


def _out_call(x, w, b):
    dout = w.shape[1]
    nb = NPAD // BNC
    return pl.pallas_call(
        _out_kernel,
        grid=(nb,),
        in_specs=[
            pl.BlockSpec((BNC, x.shape[1]), lambda i: (i, 0)),
            pl.BlockSpec((x.shape[1], dout), lambda i: (0, 0)),
            pl.BlockSpec((1, dout), lambda i: (0, 0)),
        ],
        out_specs=pl.BlockSpec((BNC, dout), lambda i: (i, 0)),
        out_shape=jax.ShapeDtypeStruct((NPAD, dout), jnp.float32),
    )(x, w, b)


def kernel(x, edge_index, W1, b1, W2, b2, W3, b3, R1, rb1, R2, rb2, R3, rb3,
           g1, be1, g2, be2, g3, be3, Wout, bout):
    n = x.shape[0]
    e = edge_index.shape[1]
    loops = jnp.arange(n, dtype=edge_index.dtype)
    src = jnp.concatenate([edge_index[0], loops])
    dst = jnp.concatenate([edge_index[1], loops])
    etot = e + n

    nch = -(-etot // (NT * CK))
    nch = -(-nch // BLK) * BLK
    epad = NT * nch * CK
    src3 = jnp.pad(src, (0, epad - etot),
                   constant_values=n).reshape(NT, nch, CK).astype(jnp.int32)
    dst3 = jnp.pad(dst, (0, epad - etot),
                   constant_values=n).reshape(NT, nch, CK).astype(jnp.int32)
    nblk = nch // BLK
    src4 = src3.reshape(NT, nblk, BLK, CK)
    dst4 = dst3.reshape(NT, nblk, BLK, CK)

    xp = jnp.pad(x, ((0, NPAD - n), (0, 0)))

    degp = _make_deg(nch)(dst3)
    deg = degp[0, :, 0] + degp[1, :, 0]
    deg_mm = deg.reshape(NPAD // BNM, 1, BNM)
    deg_cb = deg.reshape(NPAD // BNC, 1, BNC)

    agg_fn = _make_agg(nblk)

    h = xp
    layers = [(W1, b1, R1, rb1, g1, be1),
              (W2, b2, R2, rb2, g2, be2),
              (W3, b3, R3, rb3, g3, be3)]
    for (W, b, R, rb, g, be) in layers:
        hs4, res = _mm_call(h, W, R, rb[None, :], deg_mm)
        agg4 = agg_fn(hs4, src4, dst4)
        h = _combine_call(agg4, res, deg_cb, b[None, :], g[None, :],
                          be[None, :])

    out = _out_call(h, Wout, bout[None, :])
    return out[:n]
